# Initial kernel scaffold; baseline (speedup 1.0000x reference)
import jax
import jax.numpy as jnp
from jax.experimental import pallas as pl


def kernel(inputs, embedding_matrix):
    raise NotImplementedError("write your pallas kernel here")


# speedup vs baseline: 7.1217x; 7.1217x over previous
import jax
import jax.numpy as jnp
from jax.experimental import pallas as pl

MAX_CONTEXT = 200
EMBEDDING_DIM = 64
PADDING_TOKEN = 0

_BB = 128


def _body(inp_ref, emb_ref, out_ref):
    mask = (inp_ref[...] != PADDING_TOKEN).astype(jnp.float32)
    out_ref[...] = mask[:, :, None] * emb_ref[...][None, :, :]


def kernel(inputs, embedding_matrix):
    if inputs.shape[1] > MAX_CONTEXT:
        inputs = inputs[:, -MAX_CONTEXT:]
    batch, seq = inputs.shape
    dim = embedding_matrix.shape[1]
    grid = (batch // _BB,)
    return pl.pallas_call(
        _body,
        grid=grid,
        in_specs=[
            pl.BlockSpec((_BB, seq), lambda i: (i, 0)),
            pl.BlockSpec((seq, dim), lambda i: (0, 0)),
        ],
        out_specs=pl.BlockSpec((_BB, seq, dim), lambda i: (i, 0, 0)),
        out_shape=jax.ShapeDtypeStruct((batch, seq, dim), jnp.float32),
    )(inputs, embedding_matrix)


# speedup vs baseline: 9.2806x; 1.3031x over previous
import jax
import jax.numpy as jnp
from jax.experimental import pallas as pl

MAX_CONTEXT = 200
PADDING_TOKEN = 0

_BB = 256


def _body(inp_e_ref, inp_o_ref, emb_ref, out_ref):
    bb, lp = inp_e_ref.shape
    d2 = emb_ref.shape[1]
    d = d2 // 2
    m_e = (inp_e_ref[...] != PADDING_TOKEN).astype(jnp.float32)[:, :, None]
    m_o = (inp_o_ref[...] != PADDING_TOKEN).astype(jnp.float32)[:, :, None]
    mask = jnp.concatenate(
        [jnp.broadcast_to(m_e, (bb, lp, d)), jnp.broadcast_to(m_o, (bb, lp, d))],
        axis=-1,
    )
    out_ref[...] = mask * emb_ref[...][None, :, :]


def kernel(inputs, embedding_matrix):
    if inputs.shape[1] > MAX_CONTEXT:
        inputs = inputs[:, -MAX_CONTEXT:]
    batch, seq = inputs.shape
    dim = embedding_matrix.shape[1]
    lp = seq // 2
    inputs_e = inputs[:, 0::2]
    inputs_o = inputs[:, 1::2]
    emb2 = embedding_matrix.reshape(lp, 2 * dim)
    grid = (batch // _BB,)
    out2 = pl.pallas_call(
        _body,
        grid=grid,
        in_specs=[
            pl.BlockSpec((_BB, lp), lambda i: (i, 0)),
            pl.BlockSpec((_BB, lp), lambda i: (i, 0)),
            pl.BlockSpec((lp, 2 * dim), lambda i: (0, 0)),
        ],
        out_specs=pl.BlockSpec((_BB, lp, 2 * dim), lambda i: (i, 0, 0)),
        out_shape=jax.ShapeDtypeStruct((batch, lp, 2 * dim), jnp.float32),
    )(inputs_e, inputs_o, emb2)
    return out2.reshape(batch, seq, dim)


# speedup vs baseline: 9.2886x; 1.0009x over previous
import jax
import jax.numpy as jnp
from jax.experimental import pallas as pl
from jax.experimental.pallas import tpu as pltpu

MAX_CONTEXT = 200
PADDING_TOKEN = 0

_BB = 128
_K = 4


def _body(inp_e_ref, inp_o_ref, emb_ref, out_ref, buf_ref, sem):
    i = pl.program_id(0)
    n = pl.num_programs(0)
    slot = jax.lax.rem(i, 2)
    bb, lp = inp_e_ref.shape
    d2 = emb_ref.shape[1]
    d = d2 // 2
    sub = bb // _K

    def _dma(step, k, slt):
        return pltpu.make_async_copy(
            buf_ref.at[slt, pl.ds(k * sub, sub)],
            out_ref.at[pl.ds(step * bb + k * sub, sub)],
            sem.at[slt, k],
        )

    @pl.when(i >= 2)
    def _wait_prev():
        for k in range(_K):
            _dma(i - 2, k, slot).wait()

    m_e = (inp_e_ref[...] != PADDING_TOKEN).astype(jnp.float32)[:, :, None]
    m_o = (inp_o_ref[...] != PADDING_TOKEN).astype(jnp.float32)[:, :, None]
    mask = jnp.concatenate(
        [jnp.broadcast_to(m_e, (bb, lp, d)), jnp.broadcast_to(m_o, (bb, lp, d))],
        axis=-1,
    )
    buf_ref[slot] = mask * emb_ref[...][None, :, :]

    for k in range(_K):
        _dma(i, k, slot).start()

    @pl.when(i == n - 1)
    def _drain():
        for k in range(_K):
            _dma(i - 1, k, 1 - slot).wait()
            _dma(i, k, slot).wait()


def kernel(inputs, embedding_matrix):
    if inputs.shape[1] > MAX_CONTEXT:
        inputs = inputs[:, -MAX_CONTEXT:]
    batch, seq = inputs.shape
    dim = embedding_matrix.shape[1]
    lp = seq // 2
    inputs_e = inputs[:, 0::2]
    inputs_o = inputs[:, 1::2]
    emb2 = embedding_matrix.reshape(lp, 2 * dim)
    grid = (batch // _BB,)
    out2 = pl.pallas_call(
        _body,
        grid=grid,
        in_specs=[
            pl.BlockSpec((_BB, lp), lambda i: (i, 0)),
            pl.BlockSpec((_BB, lp), lambda i: (i, 0)),
            pl.BlockSpec((lp, 2 * dim), lambda i: (0, 0)),
        ],
        out_specs=pl.BlockSpec(memory_space=pl.ANY),
        out_shape=jax.ShapeDtypeStruct((batch, lp, 2 * dim), jnp.float32),
        scratch_shapes=[
            pltpu.VMEM((2, _BB, lp, 2 * dim), jnp.float32),
            pltpu.SemaphoreType.DMA((2, _K)),
        ],
    )(inputs_e, inputs_o, emb2)
    return out2.reshape(batch, seq, dim)


# speedup vs baseline: 10.4892x; 1.1293x over previous
import jax
import jax.numpy as jnp
from jax.experimental import pallas as pl
from jax.experimental.pallas import tpu as pltpu

MAX_CONTEXT = 200
PADDING_TOKEN = 0

_BB = 128
_K = 4


def _body(inp_e_ref, inp_o_ref, emb_ref, out_ref, bcast_ref, dirty_ref, sem):
    i = pl.program_id(0)
    n = pl.num_programs(0)
    slot = jax.lax.rem(i, 2)
    bb, lp = inp_e_ref.shape
    d2 = emb_ref.shape[1]
    d = d2 // 2
    sub = bb // _K

    @pl.when(i == 0)
    def _build_bcast():
        bcast_ref[...] = jnp.broadcast_to(emb_ref[...][None, :, :], (bb, lp, d2))

    def _issue(src_ref, step, k, slt):
        return pltpu.make_async_copy(
            src_ref.at[pl.ds(k * sub, sub)],
            out_ref.at[pl.ds(step * bb + k * sub, sub)],
            sem.at[slt, k],
        )

    @pl.when(i >= 2)
    def _wait_prev():
        for k in range(_K):
            _issue(bcast_ref, i - 2, k, slot).wait()

    clean = jnp.logical_and(
        jnp.all(inp_e_ref[...] != PADDING_TOKEN),
        jnp.all(inp_o_ref[...] != PADDING_TOKEN),
    )

    @pl.when(clean)
    def _fast():
        for k in range(_K):
            _issue(bcast_ref, i, k, slot).start()

    @pl.when(jnp.logical_not(clean))
    def _masked():
        m_e = (inp_e_ref[...] != PADDING_TOKEN).astype(jnp.float32)[:, :, None]
        m_o = (inp_o_ref[...] != PADDING_TOKEN).astype(jnp.float32)[:, :, None]
        mask = jnp.concatenate(
            [jnp.broadcast_to(m_e, (bb, lp, d)), jnp.broadcast_to(m_o, (bb, lp, d))],
            axis=-1,
        )
        dirty_ref[slot] = mask * emb_ref[...][None, :, :]
        for k in range(_K):
            _issue(dirty_ref.at[slot], i, k, slot).start()

    @pl.when(i == n - 1)
    def _drain():
        for k in range(_K):
            _issue(bcast_ref, i - 1, k, 1 - slot).wait()
            _issue(bcast_ref, i, k, slot).wait()


def kernel(inputs, embedding_matrix):
    if inputs.shape[1] > MAX_CONTEXT:
        inputs = inputs[:, -MAX_CONTEXT:]
    batch, seq = inputs.shape
    dim = embedding_matrix.shape[1]
    lp = seq // 2
    inputs_e = inputs[:, 0::2]
    inputs_o = inputs[:, 1::2]
    emb2 = embedding_matrix.reshape(lp, 2 * dim)
    grid = (batch // _BB,)
    out2 = pl.pallas_call(
        _body,
        grid=grid,
        in_specs=[
            pl.BlockSpec((_BB, lp), lambda i: (i, 0)),
            pl.BlockSpec((_BB, lp), lambda i: (i, 0)),
            pl.BlockSpec((lp, 2 * dim), lambda i: (0, 0)),
        ],
        out_specs=pl.BlockSpec(memory_space=pl.ANY),
        out_shape=jax.ShapeDtypeStruct((batch, lp, 2 * dim), jnp.float32),
        scratch_shapes=[
            pltpu.VMEM((_BB, lp, 2 * dim), jnp.float32),
            pltpu.VMEM((2, _BB, lp, 2 * dim), jnp.float32),
            pltpu.SemaphoreType.DMA((2, _K)),
        ],
    )(inputs_e, inputs_o, emb2)
    return out2.reshape(batch, seq, dim)
